# baseline (speedup 1.0000x reference)
import functools

import jax
import jax.numpy as jnp
from jax import lax
from jax.experimental import pallas as pl
from jax.experimental.pallas import tpu as pltpu
from jax.experimental.pallas import tpu_sc as plsc

_VOCAB = 100000
_D = 64
_B = 1024

_NC = 2
_NS = 16
_NW = _NC * _NS
_BPW = _B // _NW


@functools.cache
def _make_sc_gather():
    mesh = plsc.VectorSubcoreMesh(core_axis_name="c", subcore_axis_name="s")

    @functools.partial(
        pl.kernel,
        mesh=mesh,
        compiler_params=pltpu.CompilerParams(needs_layout_passes=False),
        out_type=jax.ShapeDtypeStruct((_B, _D), jnp.float32),
        scratch_types=[
            pltpu.VMEM((_BPW,), jnp.int32),
            pltpu.VMEM((_BPW,), jnp.int32),
            pltpu.VMEM((_BPW, 2 * _D), jnp.float32),
            pltpu.VMEM((_BPW, _D), jnp.float32),
            pltpu.SemaphoreType.DMA,
        ],
    )
    def _sc_gather(slot_hbm, par_hbm, table2_hbm, out_hbm,
                   slot_v, par_v, pair_v, row_v, sem):
        wid = lax.axis_index("s") * _NC + lax.axis_index("c")
        base = wid * _BPW
        pltpu.sync_copy(slot_hbm.at[pl.ds(base, _BPW)], slot_v)
        pltpu.sync_copy(par_hbm.at[pl.ds(base, _BPW)], par_v)
        pltpu.async_copy(table2_hbm.at[slot_v], pair_v, sem).wait()
        for g in range(_BPW // 16):
            rows16 = jax.lax.iota(jnp.int32, 16) + g * 16
            off16 = par_v[pl.ds(g * 16, 16)] * _D
            for c in range(_D):
                vals = plsc.load_gather(pair_v, [rows16, off16 + c])
                plsc.store_scatter(row_v, [rows16, jnp.full((16,), c, jnp.int32)], vals)
        pltpu.sync_copy(row_v, out_hbm.at[pl.ds(base, _BPW)])

    return _sc_gather


_MBLK = 32
_NSTEP = _B // _MBLK


def _mm_body(e_ref, wt_ref, b_ref, o_ref):
    o_ref[...] = (
        jnp.dot(e_ref[...], wt_ref[...], preferred_element_type=jnp.float32)
        + b_ref[...]
    )


def _tc_project(embeds, Wt, b2d):
    return pl.pallas_call(
        _mm_body,
        grid=(_NSTEP,),
        in_specs=[
            pl.BlockSpec((_MBLK, _D), lambda j: (j, 0)),
            pl.BlockSpec((_D, _VOCAB), lambda j: (0, 0)),
            pl.BlockSpec((1, _VOCAB), lambda j: (0, 0)),
        ],
        out_specs=pl.BlockSpec((_MBLK, _VOCAB), lambda j: (j, 0)),
        out_shape=jax.ShapeDtypeStruct((_B, _VOCAB), jnp.float32),
    )(embeds, Wt, b2d)


def kernel(x, table, W, b):
    idx = x.astype(jnp.int32)
    slot = jax.lax.shift_right_logical(idx, 1)
    par = jax.lax.bitwise_and(idx, 1)
    table2 = table.reshape(_VOCAB // 2, 2 * _D)
    embeds = _make_sc_gather()(slot, par, table2)
    return _tc_project(embeds, jnp.swapaxes(W, 0, 1), b.reshape(1, _VOCAB))


# speedup vs baseline: 1.0105x; 1.0105x over previous
import functools

import jax
import jax.numpy as jnp
from jax import lax
from jax.experimental import pallas as pl
from jax.experimental.pallas import tpu as pltpu
from jax.experimental.pallas import tpu_sc as plsc

_VOCAB = 100000
_D = 64
_B = 1024

_NC = 2
_NS = 16
_NW = _NC * _NS
_BPW = _B // _NW


@functools.cache
def _make_sc_gather():
    mesh = plsc.VectorSubcoreMesh(core_axis_name="c", subcore_axis_name="s")

    @functools.partial(
        pl.kernel,
        mesh=mesh,
        out_type=jax.ShapeDtypeStruct((_B, 2 * _D), jnp.float32),
        scratch_types=[
            pltpu.VMEM((_BPW,), jnp.int32),
            pltpu.VMEM((_BPW, 2 * _D), jnp.float32),
            pltpu.SemaphoreType.DMA,
        ],
    )
    def _sc_gather(slot_hbm, table2_hbm, out_hbm, slot_v, pair_v, sem):
        wid = lax.axis_index("s") * _NC + lax.axis_index("c")
        base = wid * _BPW
        pltpu.sync_copy(slot_hbm.at[pl.ds(base, _BPW)], slot_v)
        pltpu.async_copy(table2_hbm.at[slot_v], pair_v, sem).wait()
        pltpu.sync_copy(pair_v, out_hbm.at[pl.ds(base, _BPW)])

    return _sc_gather


_MBLK = 32
_NSTEP = _B // _MBLK


def _mm_body(e2_ref, p_ref, wt_ref, b_ref, o_ref):
    e2 = e2_ref[...]
    e = jnp.where(p_ref[...] > 0, e2[:, _D:], e2[:, :_D])
    o_ref[...] = (
        jnp.dot(e, wt_ref[...], preferred_element_type=jnp.float32)
        + b_ref[...]
    )


def _tc_project(pairs, par, Wt, b2d):
    return pl.pallas_call(
        _mm_body,
        grid=(_NSTEP,),
        in_specs=[
            pl.BlockSpec((_MBLK, 2 * _D), lambda j: (j, 0)),
            pl.BlockSpec((_MBLK, 1), lambda j: (j, 0)),
            pl.BlockSpec((_D, _VOCAB), lambda j: (0, 0)),
            pl.BlockSpec((1, _VOCAB), lambda j: (0, 0)),
        ],
        out_specs=pl.BlockSpec((_MBLK, _VOCAB), lambda j: (j, 0)),
        out_shape=jax.ShapeDtypeStruct((_B, _VOCAB), jnp.float32),
    )(pairs, par, Wt, b2d)


def kernel(x, table, W, b):
    idx = x.astype(jnp.int32)
    slot = jax.lax.shift_right_logical(idx, 1)
    par = jax.lax.bitwise_and(idx, 1).reshape(_B, 1)
    table2 = table.reshape(_VOCAB // 2, 2 * _D)
    pairs = _make_sc_gather()(slot, table2)
    return _tc_project(pairs, par, jnp.swapaxes(W, 0, 1), b.reshape(1, _VOCAB))
